# Initial kernel scaffold; baseline (speedup 1.0000x reference)
import jax
import jax.numpy as jnp
from jax.experimental import pallas as pl


def kernel(hidden_states, gate_w, w_fc, b_fc, w_proj, b_proj):
    raise NotImplementedError("write your pallas kernel here")


# speedup vs baseline: 5.7304x; 5.7304x over previous
import functools

import jax
import jax.numpy as jnp
from jax import lax
from jax.experimental import pallas as pl
from jax.experimental.pallas import tpu as pltpu
from jax.experimental.pallas import tpu_sc as plsc

E = 8
K = 2
H = 1024
I = 4096
T = 4096
TK = T * K
RBLK = 512
NB = TK // RBLK + E - 1
PAD = NB * RBLK
TB = 256
NTB = T // TB
KI = 2
IC = I // KI


def _router_kernel(x_ref, gw_ref, logits_ref, sel0_ref, sel1_ref,
                   g0_ref, g1_ref, r0_ref, r1_ref, offs_ref, cnt0_ref,
                   be_ref, carry_ref):
    g = pl.program_id(0)

    @pl.when(g == 0)
    def _():
        carry_ref[...] = jnp.zeros_like(carry_ref)

    x = x_ref[...]
    logits = lax.dot_general(x, gw_ref[...], (((1,), (1,)), ((), ())),
                             preferred_element_type=jnp.float32)
    logits_ref[...] = logits

    m = jnp.max(logits, axis=-1, keepdims=True)
    ex = jnp.exp(logits - m)
    probs = ex / jnp.sum(ex, axis=-1, keepdims=True)

    iota_e = lax.broadcasted_iota(jnp.int32, (TB, E), 1)
    m1 = jnp.max(probs, axis=-1, keepdims=True)
    i1 = jnp.min(jnp.where(probs == m1, iota_e, E), axis=-1, keepdims=True)
    probs2 = jnp.where(iota_e == i1, -1.0, probs)
    m2 = jnp.max(probs2, axis=-1, keepdims=True)
    i2 = jnp.min(jnp.where(probs2 == m2, iota_e, E), axis=-1, keepdims=True)
    s = m1 + m2
    g0_ref[...] = m1 / s
    g1_ref[...] = m2 / s
    sel0_ref[...] = i1
    sel1_ref[...] = i2

    oh0 = (lax.broadcasted_iota(jnp.int32, (TB, E), 1) == i1).astype(jnp.float32)
    oh1 = (lax.broadcasted_iota(jnp.int32, (TB, E), 1) == i2).astype(jnp.float32)
    tri_r = lax.broadcasted_iota(jnp.int32, (TB, TB), 0)
    tri_c = lax.broadcasted_iota(jnp.int32, (TB, TB), 1)
    L = (tri_r >= tri_c).astype(jnp.float32)
    cum0 = lax.dot_general(L, oh0, (((1,), (0,)), ((), ())),
                           preferred_element_type=jnp.float32)
    cum1 = lax.dot_general(L, oh1, (((1,), (0,)), ((), ())),
                           preferred_element_type=jnp.float32)
    c0 = carry_ref[0:1, :]
    c1 = carry_ref[1:2, :]
    r0 = jnp.sum(oh0 * (cum0 + c0), axis=1, keepdims=True) - 1.0
    r1 = jnp.sum(oh1 * (cum1 + c1), axis=1, keepdims=True) - 1.0
    r0_ref[...] = r0.astype(jnp.int32)
    r1_ref[...] = r1.astype(jnp.int32)
    c0n = c0 + cum0[TB - 1:TB, :]
    c1n = c1 + cum1[TB - 1:TB, :]
    carry_ref[0:1, :] = c0n
    carry_ref[1:2, :] = c1n

    @pl.when(g == pl.num_programs(0) - 1)
    def _():
        counts = c0n + c1n
        nb = jnp.floor((counts + (RBLK - 1.0)) / RBLK)
        iota_r8 = lax.broadcasted_iota(jnp.int32, (E, E), 0)
        iota_c8 = lax.broadcasted_iota(jnp.int32, (E, E), 1)
        U = (iota_r8 <= iota_c8).astype(jnp.float32)
        cumnb = lax.dot_general(nb, U, (((1,), (0,)), ((), ())),
                                preferred_element_type=jnp.float32)
        offs_ref[...] = ((cumnb - nb) * RBLK).astype(jnp.int32)
        cnt0_ref[...] = c0n.astype(jnp.int32)
        eye = (iota_r8 == iota_c8).astype(jnp.float32)
        cum_col = lax.dot_general(eye, cumnb, (((1,), (1,)), ((), ())),
                                  preferred_element_type=jnp.float32)
        b_iota = lax.broadcasted_iota(jnp.int32, (E, 32), 1).astype(jnp.float32)
        cmp = (cum_col <= b_iota).astype(jnp.float32)
        ones = jnp.ones((1, E), jnp.float32)
        be = lax.dot_general(ones, cmp, (((1,), (0,)), ((), ())),
                             preferred_element_type=jnp.float32)
        be_ref[...] = jnp.minimum(be.astype(jnp.int32), E - 1)


def _router(x, gate_w):
    return pl.pallas_call(
        _router_kernel,
        grid=(NTB,),
        in_specs=[
            pl.BlockSpec((TB, H), lambda g: (g, 0)),
            pl.BlockSpec((E, H), lambda g: (0, 0)),
        ],
        out_specs=[
            pl.BlockSpec((TB, E), lambda g: (g, 0)),
            pl.BlockSpec((TB, 1), lambda g: (g, 0)),
            pl.BlockSpec((TB, 1), lambda g: (g, 0)),
            pl.BlockSpec((TB, 1), lambda g: (g, 0)),
            pl.BlockSpec((TB, 1), lambda g: (g, 0)),
            pl.BlockSpec((TB, 1), lambda g: (g, 0)),
            pl.BlockSpec((TB, 1), lambda g: (g, 0)),
            pl.BlockSpec((1, E), lambda g: (0, 0)),
            pl.BlockSpec((1, E), lambda g: (0, 0)),
            pl.BlockSpec((1, 32), lambda g: (0, 0)),
        ],
        out_shape=[
            jax.ShapeDtypeStruct((T, E), jnp.float32),
            jax.ShapeDtypeStruct((T, 1), jnp.int32),
            jax.ShapeDtypeStruct((T, 1), jnp.int32),
            jax.ShapeDtypeStruct((T, 1), jnp.float32),
            jax.ShapeDtypeStruct((T, 1), jnp.float32),
            jax.ShapeDtypeStruct((T, 1), jnp.int32),
            jax.ShapeDtypeStruct((T, 1), jnp.int32),
            jax.ShapeDtypeStruct((1, E), jnp.int32),
            jax.ShapeDtypeStruct((1, E), jnp.int32),
            jax.ShapeDtypeStruct((1, 32), jnp.int32),
        ],
        scratch_shapes=[pltpu.VMEM((2, E), jnp.float32)],
        compiler_params=pltpu.CompilerParams(
            dimension_semantics=("arbitrary",)),
    )(x, gate_w)


_SC_MESH = plsc.VectorSubcoreMesh(core_axis_name="c", subcore_axis_name="s")


@functools.partial(
    pl.kernel, mesh=_SC_MESH,
    compiler_params=pltpu.CompilerParams(needs_layout_passes=False),
    out_type=[
        jax.ShapeDtypeStruct((TK,), jnp.int32),
        jax.ShapeDtypeStruct((PAD,), jnp.int32),
        jax.ShapeDtypeStruct((PAD,), jnp.float32),
    ],
    scratch_types=[
        pltpu.VMEM((TK,), jnp.int32),
        pltpu.VMEM((TK,), jnp.int32),
        pltpu.VMEM((TK,), jnp.float32),
        pltpu.VMEM((16,), jnp.int32),
        pltpu.VMEM((16,), jnp.int32),
        pltpu.VMEM((TK,), jnp.int32),
        pltpu.VMEM((PAD,), jnp.int32),
        pltpu.VMEM((PAD,), jnp.float32),
    ],
)
def _dispatch(sel_h, rank_h, gates_h, offs_h, cnt0_h,
              dest_h, gsrc_h, gsort_h,
              sel_v, rank_v, gates_v, offs_v, cnt0_v,
              dest_v, gsrc_v, gsort_v):
    wid = lax.axis_index("s") * 2 + lax.axis_index("c")

    @pl.when(wid == 0)
    def _():
        pltpu.sync_copy(sel_h, sel_v)
        pltpu.sync_copy(rank_h, rank_v)
        pltpu.sync_copy(gates_h, gates_v)
        pltpu.sync_copy(offs_h, offs_v)
        pltpu.sync_copy(cnt0_h, cnt0_v)

        zi = jnp.zeros((16,), jnp.int32)
        zf = jnp.zeros((16,), jnp.float32)

        def init(i, _):
            gsrc_v[pl.ds(i * 16, 16)] = zi
            gsort_v[pl.ds(i * 16, 16)] = zf
            return 0

        lax.fori_loop(0, PAD // 16, init, 0)

        lane = lax.iota(jnp.int32, 16)
        offs16 = offs_v[...]
        cnt016 = cnt0_v[...]

        def body0(i, _):
            sl = pl.ds(i * 16, 16)
            sel16 = sel_v[sl]
            dest16 = (offs16.at[sel16].get(mode="promise_in_bounds")
                      + rank_v[sl])
            dest_v[sl] = dest16
            tok = i * 16 + lane
            plsc.store_scatter(gsrc_v, [dest16], tok)
            plsc.store_scatter(gsort_v, [dest16], gates_v[sl])
            return 0

        lax.fori_loop(0, T // 16, body0, 0)

        def body1(i, _):
            sl = pl.ds(i * 16, 16)
            sel16 = sel_v[sl]
            dest16 = (offs16.at[sel16].get(mode="promise_in_bounds")
                      + cnt016.at[sel16].get(mode="promise_in_bounds")
                      + rank_v[sl])
            dest_v[sl] = dest16
            tok = i * 16 + lane - T
            plsc.store_scatter(gsrc_v, [dest16], tok)
            plsc.store_scatter(gsort_v, [dest16], gates_v[sl])
            return 0

        lax.fori_loop(T // 16, TK // 16, body1, 0)

        pltpu.sync_copy(dest_v, dest_h)
        pltpu.sync_copy(gsrc_v, gsrc_h)
        pltpu.sync_copy(gsort_v, gsort_h)


_GROWS = PAD // 32
_GCH = 16
_GNC = _GROWS // _GCH


@functools.partial(
    pl.kernel, mesh=_SC_MESH,
    out_type=jax.ShapeDtypeStruct((PAD, H), jnp.float32),
    scratch_types=[
        pltpu.VMEM((_GNC, _GCH), jnp.int32),
        pltpu.VMEM((_GCH, H), jnp.float32),
        pltpu.SemaphoreType.DMA,
    ],
)
def _gather_rows(x_h, gsrc_h, xpad_h, idx_v, rows_v, sem):
    wid = lax.axis_index("s") * 2 + lax.axis_index("c")
    base = wid * _GROWS
    pltpu.sync_copy(gsrc_h.at[wid], idx_v)

    def chunk(c, _):
        pltpu.async_copy(x_h.at[idx_v.at[c]], rows_v, sem).wait()
        pltpu.sync_copy(rows_v, xpad_h.at[pl.ds(base + c * _GCH, _GCH)])
        return 0

    lax.fori_loop(0, _GNC, chunk, 0)


def _ffn_kernel(be_ref, x_ref, wfc_ref, bfc_ref, wproj_ref, bproj_ref,
                gate_ref, out_ref):
    g = pl.program_id(0)
    ki = pl.program_id(1)
    e = be_ref[g]
    x = x_ref[...]
    h1 = lax.dot_general(x, wfc_ref[0], (((1,), (1,)), ((), ())),
                         preferred_element_type=jnp.float32)
    h1 = h1 + bfc_ref[pl.ds(e, 1), pl.ds(ki * IC, IC)]
    h1 = 0.5 * h1 * (1.0 + lax.erf(h1 * 0.7071067811865476))
    h2 = lax.dot_general(h1, wproj_ref[0], (((1,), (1,)), ((), ())),
                         preferred_element_type=jnp.float32)
    gate = gate_ref[...]
    contrib = h2 * gate

    @pl.when(ki == 0)
    def _():
        out_ref[...] = contrib + gate * bproj_ref[pl.ds(e, 1), :]

    @pl.when(ki != 0)
    def _():
        out_ref[...] += contrib


def _ffn(be, x_pad, w_fc, b_fc, w_proj, b_proj, gates_pad):
    grid_spec = pltpu.PrefetchScalarGridSpec(
        num_scalar_prefetch=1,
        grid=(NB, KI),
        in_specs=[
            pl.BlockSpec((RBLK, H), lambda g, k, be: (g, 0)),
            pl.BlockSpec((1, IC, H), lambda g, k, be: (be[g], k, 0)),
            pl.BlockSpec((E, I), lambda g, k, be: (0, 0)),
            pl.BlockSpec((1, H, IC), lambda g, k, be: (be[g], 0, k)),
            pl.BlockSpec((E, H), lambda g, k, be: (0, 0)),
            pl.BlockSpec((RBLK, 1), lambda g, k, be: (g, 0)),
        ],
        out_specs=pl.BlockSpec((RBLK, H), lambda g, k, be: (g, 0)),
    )
    return pl.pallas_call(
        _ffn_kernel,
        grid_spec=grid_spec,
        out_shape=jax.ShapeDtypeStruct((PAD, H), jnp.float32),
        compiler_params=pltpu.CompilerParams(
            dimension_semantics=("arbitrary", "arbitrary")),
    )(be, x_pad, w_fc, b_fc, w_proj, b_proj, gates_pad)


_CTOK = T // 32
_CCH = 32
_CNC = _CTOK // _CCH


@functools.partial(
    pl.kernel, mesh=_SC_MESH,
    out_type=jax.ShapeDtypeStruct((T, H), jnp.float32),
    scratch_types=[
        pltpu.VMEM((_CNC, _CCH), jnp.int32),
        pltpu.VMEM((_CNC, _CCH), jnp.int32),
        pltpu.VMEM((_CCH, H), jnp.float32),
        pltpu.VMEM((_CCH, H), jnp.float32),
        pltpu.SemaphoreType.DMA,
    ],
)
def _combine(destA_h, destB_h, hpad_h, out_h, dA_v, dB_v, bufA, bufB, sem):
    wid = lax.axis_index("s") * 2 + lax.axis_index("c")
    base = wid * _CTOK
    pltpu.sync_copy(destA_h.at[wid], dA_v)
    pltpu.sync_copy(destB_h.at[wid], dB_v)

    def chunk(c, _):
        ca = pltpu.async_copy(hpad_h.at[dA_v.at[c]], bufA, sem)
        cb = pltpu.async_copy(hpad_h.at[dB_v.at[c]], bufB, sem)
        ca.wait()
        cb.wait()

        def add(j, _):
            r = j // (H // 16)
            l = (j % (H // 16)) * 16
            bufA[r, pl.ds(l, 16)] = bufA[r, pl.ds(l, 16)] + bufB[r, pl.ds(l, 16)]
            return 0

        lax.fori_loop(0, _CCH * (H // 16), add, 0)
        pltpu.sync_copy(bufA, out_h.at[pl.ds(base + c * _CCH, _CCH)])
        return 0

    lax.fori_loop(0, _CNC, chunk, 0)


def kernel(hidden_states, gate_w, w_fc, b_fc, w_proj, b_proj):
    B, S, _ = hidden_states.shape
    x = hidden_states.reshape(T, H)
    (logits, sel0, sel1, g0, g1, r0, r1, offs, cnt0, be) = _router(x, gate_w)

    sel_flat = jnp.concatenate([sel0.reshape(T), sel1.reshape(T)])
    rank_flat = jnp.concatenate([r0.reshape(T), r1.reshape(T)])
    gates_flat = jnp.concatenate([g0.reshape(T), g1.reshape(T)])

    pad8 = jnp.zeros((8,), jnp.int32)
    dest, gsrc, gsort = _dispatch(sel_flat, rank_flat, gates_flat,
                                  jnp.concatenate([offs.reshape(E), pad8]),
                                  jnp.concatenate([cnt0.reshape(E), pad8]))
    x_pad = _gather_rows(x, gsrc.reshape(32, _GNC, _GCH))
    h_pad = _ffn(be.reshape(32)[:NB], x_pad, w_fc, b_fc, w_proj, b_proj,
                 gsort.reshape(PAD, 1))
    out = _combine(dest[:T].reshape(32, _CNC, _CCH),
                   dest[T:].reshape(32, _CNC, _CCH), h_pad)
    return out.reshape(B, S, H), logits
